# baseline (speedup 1.0000x reference)
import functools

import jax
import jax.numpy as jnp
import numpy as np
from jax import lax
from jax.experimental import pallas as pl
from jax.experimental.pallas import tpu as pltpu
from jax.experimental.pallas import tpu_sc as plsc

_NUM_PATCHES = 576
_NUM_KEEP = 144
_BATCH = 64
_DIM = 768

_UNMASKED = np.array([
    7, 10, 11, 12, 15, 16, 20, 23, 24, 25, 28, 29, 38, 44, 47, 55,
    60, 61, 68, 76, 82, 84, 87, 88, 93, 96, 111, 112, 113, 114, 119, 122,
    128, 129, 131, 135, 145, 148, 151, 152, 153, 154, 157, 168, 175, 178,
    187, 188, 199, 201, 202, 203, 209, 210, 212, 215, 217, 219, 222, 224,
    225, 229, 233, 235, 237, 238, 239, 240, 241, 245, 247, 248, 251, 255,
    257, 259, 262, 271, 278, 283, 284, 289, 290, 292, 299, 308, 313, 317,
    321, 326, 327, 332, 333, 334, 335, 339, 345, 346, 347, 356, 367, 369,
    374, 382, 383, 389, 390, 391, 393, 397, 400, 403, 413, 416, 420, 428,
    432, 434, 436, 439, 442, 444, 446, 448, 451, 454, 461, 472, 474, 478,
    486, 489, 492, 493, 495, 504, 507, 523, 528, 550, 555, 567, 569, 573,
], dtype=np.int32)

_FLAT_IDX = (
    np.arange(_BATCH, dtype=np.int32)[:, None] * _NUM_PATCHES
    + _UNMASKED[None, :]
).reshape(-1)

_NCORES = 2
_NSUB = 16
_NW = _NCORES * _NSUB
_ROWS_TOTAL = _BATCH * _NUM_KEEP
_ROWS_W = _ROWS_TOTAL // _NW
_CHUNK = 24
_NCHUNK = _ROWS_W // _CHUNK
_NBUF = 7


def _sc_gather(table, idx):
    mesh = plsc.VectorSubcoreMesh(core_axis_name="c", subcore_axis_name="s")

    @functools.partial(
        pl.kernel,
        mesh=mesh,
        out_type=jax.ShapeDtypeStruct((_ROWS_TOTAL, _DIM), jnp.float32),
        scratch_types=[
            pltpu.VMEM((_ROWS_W,), jnp.int32),
            *[pltpu.VMEM((_CHUNK, _DIM), jnp.float32) for _ in range(_NBUF)],
            *[pltpu.SemaphoreType.DMA for _ in range(2 * _NBUF)],
        ],
    )
    def k(table_hbm, idx_hbm, out_hbm, idx_v, *scratch):
        bufs = scratch[:_NBUF]
        gsems = scratch[_NBUF : 2 * _NBUF]
        wsems = scratch[2 * _NBUF :]
        wid = lax.axis_index("s") * _NCORES + lax.axis_index("c")
        base = wid * _ROWS_W

        def start_gather(c):
            b = c % _NBUF
            return pltpu.async_copy(
                table_hbm.at[idx_v.at[pl.ds(c * _CHUNK, _CHUNK)]],
                bufs[b],
                gsems[b],
            )

        pltpu.sync_copy(
            idx_hbm.at[pl.ds(base, _CHUNK)], idx_v.at[pl.ds(0, _CHUNK)]
        )
        gathers = [start_gather(0)]
        pltpu.sync_copy(
            idx_hbm.at[pl.ds(base + _CHUNK, _ROWS_W - _CHUNK)],
            idx_v.at[pl.ds(_CHUNK, _ROWS_W - _CHUNK)],
        )
        gathers += [start_gather(c) for c in range(1, min(_NBUF, _NCHUNK))]
        writes = [None] * _NCHUNK
        unwaited = set()
        for c in range(_NCHUNK):
            b = c % _NBUF
            gathers[c].wait()
            writes[c] = pltpu.async_copy(
                bufs[b], out_hbm.at[pl.ds(base + c * _CHUNK, _CHUNK)], wsems[b]
            )
            unwaited.add(c)
            nxt = c + _NBUF - 1
            if c >= 1 and nxt < _NCHUNK and len(gathers) == nxt:
                writes[c - 1].wait()
                unwaited.discard(c - 1)
                gathers.append(start_gather(nxt))
        for c in sorted(unwaited):
            writes[c].wait()

    return k(table, idx)


def kernel(patches):
    table = patches.reshape(_BATCH * _NUM_PATCHES, _DIM)
    idx = jnp.asarray(_FLAT_IDX)
    out = _sc_gather(table, idx)
    return out.reshape(_BATCH, _NUM_KEEP, _DIM)


# speedup vs baseline: 1.0086x; 1.0086x over previous
import functools

import jax
import jax.numpy as jnp
import numpy as np
from jax import lax
from jax.experimental import pallas as pl
from jax.experimental.pallas import tpu as pltpu
from jax.experimental.pallas import tpu_sc as plsc

_NUM_PATCHES = 576
_NUM_KEEP = 144
_BATCH = 64
_DIM = 768

_UNMASKED = np.array([
    7, 10, 11, 12, 15, 16, 20, 23, 24, 25, 28, 29, 38, 44, 47, 55,
    60, 61, 68, 76, 82, 84, 87, 88, 93, 96, 111, 112, 113, 114, 119, 122,
    128, 129, 131, 135, 145, 148, 151, 152, 153, 154, 157, 168, 175, 178,
    187, 188, 199, 201, 202, 203, 209, 210, 212, 215, 217, 219, 222, 224,
    225, 229, 233, 235, 237, 238, 239, 240, 241, 245, 247, 248, 251, 255,
    257, 259, 262, 271, 278, 283, 284, 289, 290, 292, 299, 308, 313, 317,
    321, 326, 327, 332, 333, 334, 335, 339, 345, 346, 347, 356, 367, 369,
    374, 382, 383, 389, 390, 391, 393, 397, 400, 403, 413, 416, 420, 428,
    432, 434, 436, 439, 442, 444, 446, 448, 451, 454, 461, 472, 474, 478,
    486, 489, 492, 493, 495, 504, 507, 523, 528, 550, 555, 567, 569, 573,
], dtype=np.int32)

_FLAT_IDX = (
    np.arange(_BATCH, dtype=np.int32)[:, None] * _NUM_PATCHES
    + _UNMASKED[None, :]
).reshape(-1)

_NCORES = 2
_NSUB = 16
_NW = _NCORES * _NSUB
_ROWS_TOTAL = _BATCH * _NUM_KEEP
_ROWS_W = _ROWS_TOTAL // _NW
_CHUNK = 24
_NCHUNK = _ROWS_W // _CHUNK
_NBUF = 7


def _sc_gather(table, idx):
    mesh = plsc.VectorSubcoreMesh(core_axis_name="c", subcore_axis_name="s")

    @functools.partial(
        pl.kernel,
        mesh=mesh,
        out_type=jax.ShapeDtypeStruct((_ROWS_TOTAL, _DIM), jnp.float32),
        scratch_types=[
            pltpu.VMEM((_ROWS_W,), jnp.int32),
            *[pltpu.VMEM((_CHUNK, _DIM), jnp.float32) for _ in range(_NBUF)],
            *[pltpu.SemaphoreType.DMA for _ in range(2 * _NBUF)],
        ],
    )
    def k(table_hbm, idx_hbm, out_hbm, idx_v, *scratch):
        bufs = scratch[:_NBUF]
        gsems = scratch[_NBUF : 2 * _NBUF]
        wsems = scratch[2 * _NBUF :]
        wid = lax.axis_index("c") * _NSUB + lax.axis_index("s")
        base = wid * _ROWS_W

        def start_gather(c):
            b = c % _NBUF
            return pltpu.async_copy(
                table_hbm.at[idx_v.at[pl.ds(c * _CHUNK, _CHUNK)]],
                bufs[b],
                gsems[b],
            )

        pltpu.sync_copy(idx_hbm.at[pl.ds(base, _ROWS_W)], idx_v)
        gathers = [start_gather(c) for c in range(min(_NBUF, _NCHUNK))]
        writes = [None] * _NCHUNK
        unwaited = set()
        for c in range(_NCHUNK):
            b = c % _NBUF
            gathers[c].wait()
            writes[c] = pltpu.async_copy(
                bufs[b], out_hbm.at[pl.ds(base + c * _CHUNK, _CHUNK)], wsems[b]
            )
            unwaited.add(c)
            nxt = c + _NBUF - 1
            if c >= 1 and nxt < _NCHUNK and len(gathers) == nxt:
                writes[c - 1].wait()
                unwaited.discard(c - 1)
                gathers.append(start_gather(nxt))
        for c in sorted(unwaited):
            writes[c].wait()

    return k(table, idx)


def kernel(patches):
    table = patches.reshape(_BATCH * _NUM_PATCHES, _DIM)
    idx = jnp.asarray(_FLAT_IDX)
    out = _sc_gather(table, idx)
    return out.reshape(_BATCH, _NUM_KEEP, _DIM)


# speedup vs baseline: 1.0220x; 1.0133x over previous
import functools

import jax
import jax.numpy as jnp
import numpy as np
from jax import lax
from jax.experimental import pallas as pl
from jax.experimental.pallas import tpu as pltpu
from jax.experimental.pallas import tpu_sc as plsc

_NUM_PATCHES = 576
_NUM_KEEP = 144
_BATCH = 64
_DIM = 768

_UNMASKED = np.array([
    7, 10, 11, 12, 15, 16, 20, 23, 24, 25, 28, 29, 38, 44, 47, 55,
    60, 61, 68, 76, 82, 84, 87, 88, 93, 96, 111, 112, 113, 114, 119, 122,
    128, 129, 131, 135, 145, 148, 151, 152, 153, 154, 157, 168, 175, 178,
    187, 188, 199, 201, 202, 203, 209, 210, 212, 215, 217, 219, 222, 224,
    225, 229, 233, 235, 237, 238, 239, 240, 241, 245, 247, 248, 251, 255,
    257, 259, 262, 271, 278, 283, 284, 289, 290, 292, 299, 308, 313, 317,
    321, 326, 327, 332, 333, 334, 335, 339, 345, 346, 347, 356, 367, 369,
    374, 382, 383, 389, 390, 391, 393, 397, 400, 403, 413, 416, 420, 428,
    432, 434, 436, 439, 442, 444, 446, 448, 451, 454, 461, 472, 474, 478,
    486, 489, 492, 493, 495, 504, 507, 523, 528, 550, 555, 567, 569, 573,
], dtype=np.int32)

_FLAT_IDX = (
    np.arange(_BATCH, dtype=np.int32)[:, None] * _NUM_PATCHES
    + _UNMASKED[None, :]
).reshape(-1)

_NCORES = 2
_NSUB = 16
_NW = _NCORES * _NSUB
_ROWS_TOTAL = _BATCH * _NUM_KEEP
_ROWS_W = _ROWS_TOTAL // _NW
_CHUNK = 24
_NCHUNK = _ROWS_W // _CHUNK
_NBUF = 7


def _sc_gather(table, idx):
    mesh = plsc.VectorSubcoreMesh(core_axis_name="c", subcore_axis_name="s")

    @functools.partial(
        pl.kernel,
        mesh=mesh,
        out_type=jax.ShapeDtypeStruct((_ROWS_TOTAL, _DIM), jnp.float32),
        scratch_types=[
            pltpu.VMEM((_ROWS_W,), jnp.int32),
            *[pltpu.VMEM((_CHUNK, _DIM), jnp.float32) for _ in range(_NBUF)],
            *[pltpu.SemaphoreType.DMA for _ in range(2 * _NBUF)],
        ],
    )
    def k(table_hbm, idx_hbm, out_hbm, idx_v, *scratch):
        bufs = scratch[:_NBUF]
        gsems = scratch[_NBUF : 2 * _NBUF]
        wsems = scratch[2 * _NBUF :]
        wid = lax.axis_index("s") * _NCORES + lax.axis_index("c")
        base = wid * _ROWS_W

        def start_gather(c):
            b = c % _NBUF
            return pltpu.async_copy(
                table_hbm.at[idx_v.at[pl.ds(c * _CHUNK, _CHUNK)]],
                bufs[b],
                gsems[b],
            )

        pltpu.sync_copy(idx_hbm.at[pl.ds(base, _ROWS_W)], idx_v)
        gathers = [start_gather(c) for c in range(min(_NBUF, _NCHUNK))]
        writes = [None] * _NCHUNK
        unwaited = set()
        for c in range(_NCHUNK):
            b = c % _NBUF
            gathers[c].wait()
            writes[c] = pltpu.async_copy(
                bufs[b], out_hbm.at[pl.ds(base + c * _CHUNK, _CHUNK)], wsems[b]
            )
            unwaited.add(c)
            nxt = c + _NBUF - 1
            if c >= 1 and nxt < _NCHUNK and len(gathers) == nxt:
                writes[c - 1].wait()
                unwaited.discard(c - 1)
                gathers.append(start_gather(nxt))
        for c in sorted(unwaited):
            writes[c].wait()

    return k(table, idx)


def kernel(patches):
    table = patches.reshape(_BATCH * _NUM_PATCHES, _DIM)
    idx = jnp.asarray(_FLAT_IDX)
    out = _sc_gather(table, idx)
    return out.reshape(_BATCH, _NUM_KEEP, _DIM)
